# baseline (speedup 1.0000x reference)
import functools

import jax
import jax.numpy as jnp
from jax import lax
from jax.experimental import pallas as pl
from jax.experimental.pallas import tpu as pltpu
from jax.experimental.pallas import tpu_sc as plsc

_N = 10000
_E = 320000
_D = 128
_NC = 2
_NS = 16
_CHUNK = 128
_CPT = 160
_EPAD = _NS * _CPT * _CHUNK
_HALF = 5600
_RPS = 360
_NACC = _RPS * _NS
_BR = 400
_BPC = _HALF // _BR


def _sc_body(pk_h, ta_h, tb_h, z_h, out,
             rowlist, pkv, garow, gbrow, dstrow, rowsa, rowsb, acc, semA, semB):
    c = lax.axis_index("c")
    s = lax.axis_index("s")
    base = c * _HALF
    pltpu.sync_copy(z_h.at[pl.ds(s * _RPS, _RPS)], acc.at[pl.ds(s * _RPS, _RPS)])
    ji = lax.iota(jnp.int32, 16)
    for g in range(2):
        for k in range(5):
            rowlist[g, pl.ds(16 * k, 16)] = ji + (s * _CPT + g * 80 + 16 * k)
        pltpu.async_copy(pk_h.at[rowlist.at[g]],
                         pkv.at[pl.ds(g * 80, 80)], semA).wait()

    plsc.subcore_barrier()

    def body(i, carry):
        for k in range(_CHUNK // 16):
            p = pkv[i, pl.ds(16 * k, 16)]
            garow[0, pl.ds(16 * k, 16)] = p & 0x3FFF
            d = ((p >> 14) & 0x3FFF) - base
            ok = (d >= 0) & (d < _HALF)
            dstrow[0, pl.ds(16 * k, 16)] = jnp.where(ok, d, _HALF)
            gbrow[0, pl.ds(16 * k, 16)] = lax.shift_right_logical(p, 28)
        c1 = pltpu.async_copy(ta_h.at[garow.at[0]], rowsa, semA)
        c2 = pltpu.async_copy(tb_h.at[gbrow.at[0]], rowsb, semB)
        c1.wait()
        pltpu.sync_copy(rowsa, acc.at[dstrow.at[0]], add=True)
        c2.wait()
        pltpu.sync_copy(rowsb, acc.at[dstrow.at[0]], add=True)
        return carry

    lax.fori_loop(0, _CPT, body, 0)
    plsc.subcore_barrier()
    pltpu.sync_copy(acc.at[pl.ds(s * _RPS, _RPS)],
                    out.at[pl.ds(c * _NACC + s * _RPS, _RPS)])


@functools.lru_cache(maxsize=1)
def _sc_kernels():
    mesh = plsc.VectorSubcoreMesh(core_axis_name="c", subcore_axis_name="s")
    spmm = functools.partial(
        pl.kernel,
        mesh=mesh,
        out_type=jax.ShapeDtypeStruct((_NC * _NACC, _D), jnp.float32),
        scratch_types=[
            pltpu.VMEM((2, 80), jnp.int32),
            pltpu.VMEM((_CPT, _CHUNK), jnp.int32),
            pltpu.VMEM((1, _CHUNK), jnp.int32),
            pltpu.VMEM((1, _CHUNK), jnp.int32),
            pltpu.VMEM((1, _CHUNK), jnp.int32),
            pltpu.VMEM((_CHUNK, _D), jnp.float32),
            pltpu.VMEM((_CHUNK, _D), jnp.float32),
            pltpu.VMEM_SHARED((_NACC, _D), jnp.float32),
            pltpu.SemaphoreType.DMA,
            pltpu.SemaphoreType.DMA,
        ],
    )(_sc_body)
    return spmm


def _cl_body(h_ref, w_ref, ei_ref, b1_ref, e1_ref, b2_ref, e2_ref,
             lin_ref, hEneg_ref):
    csrc = ei_ref[0, :][:, None]
    cdst = ei_ref[1, :][:, None]
    ew = w_ref[0, :]

    def seg8(v, idx):
        rows = []
        for t in range(8):
            m = idx == t
            rows.append(jnp.sum(jnp.where(m, v, 0.0), axis=0, keepdims=True))
        return jnp.concatenate(rows, axis=0)

    def gath(tab, idx):
        o = jnp.zeros((64, tab.shape[1]), jnp.float32)
        for t in range(8):
            m = idx == t
            o = o + jnp.where(m, tab[t:t + 1, :], 0.0)
        return o

    def agnn(feat, beta, eps):
        n = jnp.sqrt(jnp.sum(feat * feat, axis=1, keepdims=True))
        nh = feat / jnp.maximum(n, 1e-12)
        e2 = (beta * ew)[:, None]
        em = []
        for t in range(8):
            m = csrc == t
            em.append(jnp.max(jnp.where(m, e2, -jnp.inf), axis=0, keepdims=True))
        em8 = jnp.concatenate(em, axis=0)
        es = jnp.exp(e2 - gath(em8, csrc))
        p = es / jnp.maximum(gath(seg8(es, csrc), csrc), 1e-12)
        m64 = gath(nh, csrc) * p
        agg = seg8(m64, cdst)
        return (1.0 + eps) * feat + agg

    h = jnp.maximum(agnn(h_ref[...], b1_ref[0, 0], e1_ref[0, 0]), 0.0)
    h = jnp.maximum(agnn(h, b2_ref[0, 0], e2_ref[0, 0]), 0.0)
    hEneg_ref[...] = -jnp.dot(h, lin_ref[...],
                              preferred_element_type=jnp.float32)


def _dense1_body(f_ref, s_ref, w1_ref, w2_ref, o_ref):
    x = (jnp.dot(f_ref[...], w1_ref[...], preferred_element_type=jnp.float32)
         + jnp.dot(s_ref[0], w2_ref[...], preferred_element_type=jnp.float32))
    o_ref[...] = jnp.tanh(x)


def _dense2_body(x_ref, s_ref, w1_ref, w2_ref, wp_ref, bp_ref, o_ref):
    x2 = jnp.tanh(
        jnp.dot(x_ref[...], w1_ref[...], preferred_element_type=jnp.float32)
        + jnp.dot(s_ref[0], w2_ref[...], preferred_element_type=jnp.float32))
    o_ref[...] = jnp.dot(x2, wp_ref[...],
                         preferred_element_type=jnp.float32) + bp_ref[...]


def _agg_spec():
    return pl.BlockSpec((1, _BR, _D), lambda i: (i // _BPC, i % _BPC, 0))


def _dense_layer1(feats, s1, W1, W2):
    return pl.pallas_call(
        _dense1_body,
        grid=(_N // _BR,),
        in_specs=[
            pl.BlockSpec((_BR, _D), lambda i: (i, 0)),
            _agg_spec(),
            pl.BlockSpec((_D, _D), lambda i: (0, 0)),
            pl.BlockSpec((_D, _D), lambda i: (0, 0)),
        ],
        out_specs=pl.BlockSpec((_BR, _D), lambda i: (i, 0)),
        out_shape=jax.ShapeDtypeStruct((_N, _D), jnp.float32),
    )(feats, s1, W1, W2)


def _dense_layer2(x1, s2, W1, W2, Wp, bp):
    return pl.pallas_call(
        _dense2_body,
        grid=(_N // _BR,),
        in_specs=[
            pl.BlockSpec((_BR, _D), lambda i: (i, 0)),
            _agg_spec(),
            pl.BlockSpec((_D, _D), lambda i: (0, 0)),
            pl.BlockSpec((_D, _D), lambda i: (0, 0)),
            pl.BlockSpec((_D, 16), lambda i: (0, 0)),
            pl.BlockSpec((1, 16), lambda i: (0, 0)),
        ],
        out_specs=pl.BlockSpec((_BR, 16), lambda i: (i, 0)),
        out_shape=jax.ShapeDtypeStruct((_N, 16), jnp.float32),
    )(x1, s2, W1, W2, Wp, bp)


def kernel(cl_h, cl_w, cl_edge_index, edge_index, etype, feats_W, linE_W,
           beta1, eps1, beta2, eps2, W1a, W2a, W1b, W2b, Wp, bp):
    f32 = jnp.float32
    hEneg = pl.pallas_call(
        _cl_body,
        out_shape=jax.ShapeDtypeStruct((8, _D), f32),
    )(cl_h, cl_w.reshape(1, 64), cl_edge_index.astype(jnp.int32),
      beta1.reshape(1, 1), eps1.reshape(1, 1),
      beta2.reshape(1, 1), eps2.reshape(1, 1),
      linE_W)

    src = edge_index[0].astype(jnp.int32)
    dst = edge_index[1].astype(jnp.int32)
    et = etype.astype(jnp.int32)
    pad = _EPAD - _E
    src_p = jnp.concatenate([src, jnp.zeros((pad,), jnp.int32)])
    dst_p = jnp.concatenate([dst, jnp.full((pad,), _N, jnp.int32)])
    et_p = jnp.concatenate([et, jnp.zeros((pad,), jnp.int32)])
    pk = (src_p | (dst_p << 14) | (et_p << 28)).reshape(_NS * _CPT, _CHUNK)
    z128 = jnp.zeros((_NACC, _D), f32)

    spmm = _sc_kernels()
    agg1 = spmm(pk, feats_W, hEneg, z128).reshape(_NC, _NACC, _D)
    x1 = _dense_layer1(feats_W, agg1, W1a, W2a)
    agg2 = spmm(pk, x1, hEneg, z128).reshape(_NC, _NACC, _D)
    return _dense_layer2(x1, agg2, W1b, W2b, Wp, bp.reshape(1, 16))


# speedup vs baseline: 1.0023x; 1.0023x over previous
import functools

import jax
import jax.numpy as jnp
from jax import lax
from jax.experimental import pallas as pl
from jax.experimental.pallas import tpu as pltpu
from jax.experimental.pallas import tpu_sc as plsc

_N = 10000
_E = 320000
_D = 128
_NC = 2
_NS = 16
_CHUNK = 128
_CPT = 160
_EPAD = _NS * _CPT * _CHUNK
_HALF = 5600
_RPS = 360
_NACC = _RPS * _NS
_BR = 400
_BPC = _HALF // _BR


def _sc_body(pk_h, ta_h, tb_h, z_h, out,
             pkrow, gav, gbv, dstv, ra0, ra1, rb0, rb1, acc,
             sga0, sga1, sgb0, sgb1, ssa0, ssa1, ssb0, ssb1):
    c = lax.axis_index("c")
    s = lax.axis_index("s")
    base = c * _HALF
    ra = (ra0, ra1)
    rb = (rb0, rb1)
    sga = (sga0, sga1)
    sgb = (sgb0, sgb1)
    ssa = (ssa0, ssa1)
    ssb = (ssb0, ssb1)

    pltpu.sync_copy(z_h.at[pl.ds(s * _RPS, _RPS)], acc.at[pl.ds(s * _RPS, _RPS)])
    plsc.subcore_barrier()

    def stage(i, b):
        pltpu.sync_copy(pk_h.at[pl.ds(s * _CPT + i, 1)], pkrow.at[pl.ds(b, 1)])
        for k in range(_CHUNK // 16):
            p = pkrow[b, pl.ds(16 * k, 16)]
            gav[b, pl.ds(16 * k, 16)] = p & 0x3FFF
            d = ((p >> 14) & 0x3FFF) - base
            ok = (d >= 0) & (d < _HALF)
            dstv[b, pl.ds(16 * k, 16)] = jnp.where(ok, d, _HALF)
            gbv[b, pl.ds(16 * k, 16)] = lax.shift_right_logical(p, 28)
        pltpu.async_copy(ta_h.at[gav.at[b]], ra[b], sga[b])
        pltpu.async_copy(tb_h.at[gbv.at[b]], rb[b], sgb[b])

    def wait_gathers(b):
        pltpu.make_async_copy(ta_h.at[gav.at[b]], ra[b], sga[b]).wait()
        pltpu.make_async_copy(tb_h.at[gbv.at[b]], rb[b], sgb[b]).wait()

    def fire_scatters(b):
        pltpu.async_copy(ra[b], acc.at[dstv.at[b]], ssa[b], add=True)
        pltpu.async_copy(rb[b], acc.at[dstv.at[b]], ssb[b], add=True)

    def wait_scatters(b):
        pltpu.make_async_copy(ra[b], acc.at[dstv.at[b]], ssa[b]).wait()
        pltpu.make_async_copy(rb[b], acc.at[dstv.at[b]], ssb[b]).wait()

    stage(0, 0)
    stage(1, 1)

    def body(g, carry):
        i = 2 * g
        wait_gathers(0)
        fire_scatters(0)
        wait_gathers(1)
        fire_scatters(1)
        wait_scatters(0)
        stage(i + 2, 0)
        wait_scatters(1)
        stage(i + 3, 1)
        return carry

    lax.fori_loop(0, _CPT // 2 - 1, body, 0)
    for b in (0, 1):
        wait_gathers(b)
        fire_scatters(b)
    for b in (0, 1):
        wait_scatters(b)
    plsc.subcore_barrier()
    pltpu.sync_copy(acc.at[pl.ds(s * _RPS, _RPS)],
                    out.at[pl.ds(c * _NACC + s * _RPS, _RPS)])


@functools.lru_cache(maxsize=1)
def _sc_kernels():
    mesh = plsc.VectorSubcoreMesh(core_axis_name="c", subcore_axis_name="s")
    spmm = functools.partial(
        pl.kernel,
        mesh=mesh,
        out_type=jax.ShapeDtypeStruct((_NC * _NACC, _D), jnp.float32),
        scratch_types=[
            pltpu.VMEM((2, _CHUNK), jnp.int32),
            pltpu.VMEM((2, _CHUNK), jnp.int32),
            pltpu.VMEM((2, _CHUNK), jnp.int32),
            pltpu.VMEM((2, _CHUNK), jnp.int32),
            pltpu.VMEM((_CHUNK, _D), jnp.float32),
            pltpu.VMEM((_CHUNK, _D), jnp.float32),
            pltpu.VMEM((_CHUNK, _D), jnp.float32),
            pltpu.VMEM((_CHUNK, _D), jnp.float32),
            pltpu.VMEM_SHARED((_NACC, _D), jnp.float32),
            pltpu.SemaphoreType.DMA,
            pltpu.SemaphoreType.DMA,
            pltpu.SemaphoreType.DMA,
            pltpu.SemaphoreType.DMA,
            pltpu.SemaphoreType.DMA,
            pltpu.SemaphoreType.DMA,
            pltpu.SemaphoreType.DMA,
            pltpu.SemaphoreType.DMA,
        ],
    )(_sc_body)
    return spmm


def _cl_body(h_ref, w_ref, ei_ref, b1_ref, e1_ref, b2_ref, e2_ref,
             lin_ref, hEneg_ref):
    csrc = ei_ref[0, :][:, None]
    cdst = ei_ref[1, :][:, None]
    ew = w_ref[0, :]

    def seg8(v, idx):
        rows = []
        for t in range(8):
            m = idx == t
            rows.append(jnp.sum(jnp.where(m, v, 0.0), axis=0, keepdims=True))
        return jnp.concatenate(rows, axis=0)

    def gath(tab, idx):
        o = jnp.zeros((64, tab.shape[1]), jnp.float32)
        for t in range(8):
            m = idx == t
            o = o + jnp.where(m, tab[t:t + 1, :], 0.0)
        return o

    def agnn(feat, beta, eps):
        n = jnp.sqrt(jnp.sum(feat * feat, axis=1, keepdims=True))
        nh = feat / jnp.maximum(n, 1e-12)
        e2 = (beta * ew)[:, None]
        em = []
        for t in range(8):
            m = csrc == t
            em.append(jnp.max(jnp.where(m, e2, -jnp.inf), axis=0, keepdims=True))
        em8 = jnp.concatenate(em, axis=0)
        es = jnp.exp(e2 - gath(em8, csrc))
        p = es / jnp.maximum(gath(seg8(es, csrc), csrc), 1e-12)
        m64 = gath(nh, csrc) * p
        agg = seg8(m64, cdst)
        return (1.0 + eps) * feat + agg

    h = jnp.maximum(agnn(h_ref[...], b1_ref[0, 0], e1_ref[0, 0]), 0.0)
    h = jnp.maximum(agnn(h, b2_ref[0, 0], e2_ref[0, 0]), 0.0)
    hEneg_ref[...] = -jnp.dot(h, lin_ref[...],
                              preferred_element_type=jnp.float32)


def _dense1_body(f_ref, s_ref, w1_ref, w2_ref, o_ref):
    x = (jnp.dot(f_ref[...], w1_ref[...], preferred_element_type=jnp.float32)
         + jnp.dot(s_ref[0], w2_ref[...], preferred_element_type=jnp.float32))
    o_ref[...] = jnp.tanh(x)


def _dense2_body(x_ref, s_ref, w1_ref, w2_ref, wp_ref, bp_ref, o_ref):
    x2 = jnp.tanh(
        jnp.dot(x_ref[...], w1_ref[...], preferred_element_type=jnp.float32)
        + jnp.dot(s_ref[0], w2_ref[...], preferred_element_type=jnp.float32))
    o_ref[...] = jnp.dot(x2, wp_ref[...],
                         preferred_element_type=jnp.float32) + bp_ref[...]


def _agg_spec():
    return pl.BlockSpec((1, _BR, _D), lambda i: (i // _BPC, i % _BPC, 0))


def _dense_layer1(feats, s1, W1, W2):
    return pl.pallas_call(
        _dense1_body,
        grid=(_N // _BR,),
        in_specs=[
            pl.BlockSpec((_BR, _D), lambda i: (i, 0)),
            _agg_spec(),
            pl.BlockSpec((_D, _D), lambda i: (0, 0)),
            pl.BlockSpec((_D, _D), lambda i: (0, 0)),
        ],
        out_specs=pl.BlockSpec((_BR, _D), lambda i: (i, 0)),
        out_shape=jax.ShapeDtypeStruct((_N, _D), jnp.float32),
    )(feats, s1, W1, W2)


def _dense_layer2(x1, s2, W1, W2, Wp, bp):
    return pl.pallas_call(
        _dense2_body,
        grid=(_N // _BR,),
        in_specs=[
            pl.BlockSpec((_BR, _D), lambda i: (i, 0)),
            _agg_spec(),
            pl.BlockSpec((_D, _D), lambda i: (0, 0)),
            pl.BlockSpec((_D, _D), lambda i: (0, 0)),
            pl.BlockSpec((_D, 16), lambda i: (0, 0)),
            pl.BlockSpec((1, 16), lambda i: (0, 0)),
        ],
        out_specs=pl.BlockSpec((_BR, 16), lambda i: (i, 0)),
        out_shape=jax.ShapeDtypeStruct((_N, 16), jnp.float32),
    )(x1, s2, W1, W2, Wp, bp)


def kernel(cl_h, cl_w, cl_edge_index, edge_index, etype, feats_W, linE_W,
           beta1, eps1, beta2, eps2, W1a, W2a, W1b, W2b, Wp, bp):
    f32 = jnp.float32
    hEneg = pl.pallas_call(
        _cl_body,
        out_shape=jax.ShapeDtypeStruct((8, _D), f32),
    )(cl_h, cl_w.reshape(1, 64), cl_edge_index.astype(jnp.int32),
      beta1.reshape(1, 1), eps1.reshape(1, 1),
      beta2.reshape(1, 1), eps2.reshape(1, 1),
      linE_W)

    src = edge_index[0].astype(jnp.int32)
    dst = edge_index[1].astype(jnp.int32)
    et = etype.astype(jnp.int32)
    pad = _EPAD - _E
    src_p = jnp.concatenate([src, jnp.zeros((pad,), jnp.int32)])
    dst_p = jnp.concatenate([dst, jnp.full((pad,), _N, jnp.int32)])
    et_p = jnp.concatenate([et, jnp.zeros((pad,), jnp.int32)])
    pk = (src_p | (dst_p << 14) | (et_p << 28)).reshape(_NS * _CPT, _CHUNK)
    z128 = jnp.zeros((_NACC, _D), f32)

    spmm = _sc_kernels()
    agg1 = spmm(pk, feats_W, hEneg, z128).reshape(_NC, _NACC, _D)
    x1 = _dense_layer1(feats_W, agg1, W1a, W2a)
    agg2 = spmm(pk, x1, hEneg, z128).reshape(_NC, _NACC, _D)
    return _dense_layer2(x1, agg2, W1b, W2b, Wp, bp.reshape(1, 16))
